# baseline (speedup 1.0000x reference)
import jax
import jax.numpy as jnp
from jax.experimental import pallas as pl
from jax.experimental.pallas import tpu as pltpu

_D = 256
_H = 256
_BLK = 512
_HALF = 4096


def _gcn_body(f1_ref, adj1_ref, f2_ref, adj2_ref, w1_ref, b1_ref, wg_ref,
              wih_ref, whh_ref, bih_ref, bhh_ref, out1_ref, out2_ref,
              conv_ref, gi_ref):
    pid = pl.program_id(0)

    f = f1_ref[:]
    xw = jnp.dot(f, w1_ref[:], preferred_element_type=jnp.float32)
    conv_ref[:] = xw

    @pl.when(pid == 0)
    def _():
        nrm = jnp.sqrt(jnp.sum(f * f, axis=1, keepdims=True))
        gram = jax.lax.dot_general(
            f, f, (((1,), (1,)), ((), ())),
            preferred_element_type=jnp.float32,
            precision=jax.lax.Precision.HIGHEST)
        denom = jnp.maximum(nrm * jnp.transpose(nrm), 1e-8)
        sim = gram / denom
        mn = jnp.min(sim)
        mx = jnp.max(sim)
        simn = (sim - mn) / (mx - mn)
        deg = jnp.sum(simn, axis=1, keepdims=True)
        dinv = jnp.where(deg > 0, deg ** -0.5, 0.0)
        p = dinv * xw
        q = jnp.dot(simn, p, preferred_element_type=jnp.float32,
                    precision=jax.lax.Precision.HIGHEST)
        conv_ref[:] = dinv * q

    x = jnp.maximum(conv_ref[:] + b1_ref[:], 0.0)
    adj = adj1_ref[:]
    smean = jnp.mean(adj, axis=1, keepdims=True)
    smax = jnp.max(adj, axis=1, keepdims=True)
    x = jnp.maximum(x * (1.0 + smean + smax), 0.0)

    h = x
    for i in range(2):
        gi_ref[:] = bih_ref[:]

        @pl.when(pid == 0)
        def _():
            m = jnp.dot(h, wg_ref[i], preferred_element_type=jnp.float32)
            aggvec = jnp.sum(m, axis=0, keepdims=True)
            gi_ref[:] = jnp.dot(
                aggvec, wih_ref[:],
                preferred_element_type=jnp.float32) + bih_ref[:]

        gi = gi_ref[:]
        gh = jnp.dot(h, whh_ref[:],
                     preferred_element_type=jnp.float32) + bhh_ref[:]
        r = jax.nn.sigmoid(gi[:, :_H] + gh[:, :_H])
        z = jax.nn.sigmoid(gi[:, _H:2 * _H] + gh[:, _H:2 * _H])
        n = jnp.tanh(gi[:, 2 * _H:] + r * gh[:, 2 * _H:])
        h = (1.0 - z) * n + z * h

    out1_ref[:] = jnp.maximum(h, 0.0)

    f = f2_ref[:]
    xw = jnp.dot(f, w1_ref[:], preferred_element_type=jnp.float32)
    x = jnp.maximum(xw + b1_ref[:], 0.0)
    adj = adj2_ref[:]
    smean = jnp.mean(adj, axis=1, keepdims=True)
    smax = jnp.max(adj, axis=1, keepdims=True)
    x = jnp.maximum(x * (1.0 + smean + smax), 0.0)

    h = x
    gi = bih_ref[:]
    for _ in range(2):
        gh = jnp.dot(h, whh_ref[:],
                     preferred_element_type=jnp.float32) + bhh_ref[:]
        r = jax.nn.sigmoid(gi[:, :_H] + gh[:, :_H])
        z = jax.nn.sigmoid(gi[:, _H:2 * _H] + gh[:, _H:2 * _H])
        n = jnp.tanh(gi[:, 2 * _H:] + r * gh[:, 2 * _H:])
        h = (1.0 - z) * n + z * h

    out2_ref[:] = jnp.maximum(h, 0.0)


@jax.jit
def kernel(input1, input2, adj_sem_ori, adj_sem_gcn, W1, b1, Wg, W_ih, W_hh,
           b_ih, b_hh):
    b, s, d = input1.shape
    blk = pl.BlockSpec((_BLK, _D), lambda i: (i, 0))
    const2 = lambda i: (0, 0)
    out1, out2 = pl.pallas_call(
        _gcn_body,
        grid=(_HALF // _BLK,),
        in_specs=[
            blk, blk, blk, blk,
            pl.BlockSpec((_D, _H), const2),
            pl.BlockSpec((1, _H), const2),
            pl.BlockSpec((2, _H, _H), lambda i: (0, 0, 0)),
            pl.BlockSpec((_H, 3 * _H), const2),
            pl.BlockSpec((_H, 3 * _H), const2),
            pl.BlockSpec((1, 3 * _H), const2),
            pl.BlockSpec((1, 3 * _H), const2),
        ],
        out_specs=[pl.BlockSpec((_BLK, _H), lambda i: (i, 0)),
                   pl.BlockSpec((_BLK, _H), lambda i: (i, 0))],
        out_shape=[jax.ShapeDtypeStruct((_HALF, _H), jnp.float32),
                   jax.ShapeDtypeStruct((_HALF, _H), jnp.float32)],
        scratch_shapes=[pltpu.VMEM((_BLK, _H), jnp.float32),
                        pltpu.VMEM((1, 3 * _H), jnp.float32)],
        compiler_params=pltpu.CompilerParams(
            dimension_semantics=("parallel",)),
    )(input1.reshape(-1, d), adj_sem_ori.reshape(-1, s),
      input2.reshape(-1, d), adj_sem_gcn.reshape(-1, s),
      W1, b1.reshape(1, -1), Wg, W_ih.T, W_hh.T,
      b_ih.reshape(1, -1), b_hh.reshape(1, -1))
    return (out1.reshape(b, s, _H), out2.reshape(b, s, _H))


# speedup vs baseline: 1.0073x; 1.0073x over previous
import jax
import jax.numpy as jnp
from jax.experimental import pallas as pl
from jax.experimental.pallas import tpu as pltpu

_D = 256
_H = 256
_BLK = 512
_HALF = 4096


def _gcn_body(f1_ref, adj1_ref, f2_ref, adj2_ref, w1_ref, b1_ref, wg_ref,
              wih_ref, whh_ref, bih_ref, bhh_ref, out1_ref, out2_ref,
              conv_ref, gi_ref):
    pid = pl.program_id(0)

    f = f1_ref[:]
    xw = jnp.dot(f, w1_ref[:], preferred_element_type=jnp.float32)
    conv_ref[:] = xw

    @pl.when(pid == 0)
    def _():
        nrm = jnp.sqrt(jnp.sum(f * f, axis=1, keepdims=True))
        gram = jax.lax.dot_general(
            f, f, (((1,), (1,)), ((), ())),
            preferred_element_type=jnp.float32,
            precision=jax.lax.Precision.HIGHEST)
        denom = jnp.maximum(nrm * jnp.transpose(nrm), 1e-8)
        sim = gram / denom
        mn = jnp.min(sim)
        mx = jnp.max(sim)
        simn = (sim - mn) / (mx - mn)
        deg = jnp.sum(simn, axis=1, keepdims=True)
        dinv = jnp.where(deg > 0, deg ** -0.5, 0.0)
        p = dinv * xw
        q = jnp.dot(simn, p, preferred_element_type=jnp.float32,
                    precision=jax.lax.Precision.HIGHEST)
        conv_ref[:] = dinv * q

    def pool_scale(adj):
        smean = jnp.mean(adj, axis=1, keepdims=True)
        smax = jnp.max(adj, axis=1, keepdims=True)
        return 1.0 + smean + smax

    x1 = jnp.maximum(conv_ref[:] + b1_ref[:], 0.0)
    x1 = jnp.maximum(x1 * pool_scale(adj1_ref[:]), 0.0)

    f2 = f2_ref[:]
    xw2 = jnp.dot(f2, w1_ref[:], preferred_element_type=jnp.float32)
    x2 = jnp.maximum(xw2 + b1_ref[:], 0.0)
    x2 = jnp.maximum(x2 * pool_scale(adj2_ref[:]), 0.0)

    def gru(h, gi, gh):
        r = jax.nn.sigmoid(gi[:, :_H] + gh[:, :_H])
        z = jax.nn.sigmoid(gi[:, _H:2 * _H] + gh[:, _H:2 * _H])
        n = jnp.tanh(gi[:, 2 * _H:] + r * gh[:, 2 * _H:])
        return (1.0 - z) * n + z * h

    h1, h2 = x1, x2
    for i in range(2):
        gi_ref[:] = bih_ref[:]

        @pl.when(pid == 0)
        def _():
            m = jnp.dot(h1, wg_ref[i], preferred_element_type=jnp.float32)
            aggvec = jnp.sum(m, axis=0, keepdims=True)
            gi_ref[:] = jnp.dot(
                aggvec, wih_ref[:],
                preferred_element_type=jnp.float32) + bih_ref[:]

        gh1 = jnp.dot(h1, whh_ref[:],
                      preferred_element_type=jnp.float32) + bhh_ref[:]
        gh2 = jnp.dot(h2, whh_ref[:],
                      preferred_element_type=jnp.float32) + bhh_ref[:]
        h1 = gru(h1, gi_ref[:], gh1)
        h2 = gru(h2, bih_ref[:], gh2)

    out1_ref[:] = jnp.maximum(h1, 0.0)
    out2_ref[:] = jnp.maximum(h2, 0.0)


@jax.jit
def kernel(input1, input2, adj_sem_ori, adj_sem_gcn, W1, b1, Wg, W_ih, W_hh,
           b_ih, b_hh):
    b, s, d = input1.shape
    blk = pl.BlockSpec((_BLK, _D), lambda i: (i, 0))
    const2 = lambda i: (0, 0)
    out1, out2 = pl.pallas_call(
        _gcn_body,
        grid=(_HALF // _BLK,),
        in_specs=[
            blk, blk, blk, blk,
            pl.BlockSpec((_D, _H), const2),
            pl.BlockSpec((1, _H), const2),
            pl.BlockSpec((2, _H, _H), lambda i: (0, 0, 0)),
            pl.BlockSpec((_H, 3 * _H), const2),
            pl.BlockSpec((_H, 3 * _H), const2),
            pl.BlockSpec((1, 3 * _H), const2),
            pl.BlockSpec((1, 3 * _H), const2),
        ],
        out_specs=[pl.BlockSpec((_BLK, _H), lambda i: (i, 0)),
                   pl.BlockSpec((_BLK, _H), lambda i: (i, 0))],
        out_shape=[jax.ShapeDtypeStruct((_HALF, _H), jnp.float32),
                   jax.ShapeDtypeStruct((_HALF, _H), jnp.float32)],
        scratch_shapes=[pltpu.VMEM((_BLK, _H), jnp.float32),
                        pltpu.VMEM((1, 3 * _H), jnp.float32)],
        compiler_params=pltpu.CompilerParams(
            dimension_semantics=("parallel",)),
    )(input1.reshape(-1, d), adj_sem_ori.reshape(-1, s),
      input2.reshape(-1, d), adj_sem_gcn.reshape(-1, s),
      W1, b1.reshape(1, -1), Wg, W_ih.T, W_hh.T,
      b_ih.reshape(1, -1), b_hh.reshape(1, -1))
    return (out1.reshape(b, s, _H), out2.reshape(b, s, _H))
